# baseline (speedup 1.0000x reference)
import jax
import jax.numpy as jnp
from jax import lax
from jax.experimental import pallas as pl
from jax.experimental.pallas import tpu as pltpu
from jax.experimental.pallas import tpu_sc as plsc

B, S, D = 1024, 200, 64
VOCAB = 1000000
N = B * S
NC, NS = 2, 16
NW = NC * NS
NCHUNK = B // NW
CHUNK = S
G1 = 128
G2 = CHUNK - G1
EPS = 1e-12
NLANE = 16
NVR = D // NLANE
UNROLL = 8
BKV = 2048


def _srsqrt(v):
    i = lax.bitcast_convert_type(v, jnp.int32)
    y = lax.bitcast_convert_type(jnp.int32(0x5F3759DF) - (i >> 1), jnp.float32)
    hv = jnp.float32(0.5) * v
    for _ in range(3):
        y = y * (jnp.float32(1.5) - hv * y * y)
    return y


NSB = (VOCAB + 2 * BKV - 1) // (2 * BKV)
VROWS = NSB * BKV


def _repack_body(wt1_ref, wt2_ref, out_ref):
    eye = jnp.eye(D, dtype=jnp.float32)
    dn = (((0,), (0,)), ((), ()))
    t1 = jax.lax.dot_general(wt1_ref[...], eye, dn,
                             precision=jax.lax.Precision.HIGHEST)
    t2 = jax.lax.dot_general(wt2_ref[...], eye, dn,
                             precision=jax.lax.Precision.HIGHEST)
    out_ref[...] = jnp.concatenate([t1, t2], axis=1)


@jax.jit
def _tc_repack(Wt):
    return pl.pallas_call(
        _repack_body,
        grid=(NSB,),
        in_specs=[pl.BlockSpec((D, BKV), lambda i: (0, 2 * i)),
                  pl.BlockSpec(
                      (D, BKV),
                      lambda i: (0, jnp.minimum(2 * i + 1,
                                                (VOCAB - 1) // BKV)))],
        out_specs=pl.BlockSpec((BKV, 2 * D), lambda i: (i, 0)),
        out_shape=jax.ShapeDtypeStruct((VROWS, 2 * D), jnp.float32),
    )(Wt, Wt)


def _body(idxp_hbm, hoff_hbm, table2_hbm, pos_hbm, gamma_hbm, beta_hbm,
          out3_hbm, idxp_v, hoff_v, pos_v, pbufs, obufs, gam_v, bet_v,
          gsems, ssems):
    wid = lax.axis_index("s") * NC + lax.axis_index("c")
    brow = wid * NCHUNK
    pltpu.sync_copy(idxp_hbm.at[pl.ds(brow, NCHUNK)], idxp_v)
    pltpu.sync_copy(hoff_hbm.at[pl.ds(brow, NCHUNK)],
                    hoff_v.at[pl.ds(0, NCHUNK)])
    pltpu.sync_copy(pos_hbm, pos_v)
    pltpu.sync_copy(gamma_hbm, gam_v)
    pltpu.sync_copy(beta_hbm, bet_v)
    g = [gam_v[pl.ds(NLANE * k, NLANE)] for k in range(NVR)]
    bt = [bet_v[pl.ds(NLANE * k, NLANE)] for k in range(NVR)]
    inv_d = jnp.float32(1.0 / D)

    def gather(c, b):
        return (
            pltpu.make_async_copy(
                table2_hbm.at[idxp_v.at[c, pl.ds(0, G1)]],
                pbufs[b].at[pl.ds(0, G1)], gsems[b]),
            pltpu.make_async_copy(
                table2_hbm.at[idxp_v.at[c, pl.ds(G1, G2)]],
                pbufs[b].at[pl.ds(G1, G2)], gsems[b]),
        )

    def store(c, b):
        return pltpu.make_async_copy(obufs[b], out3_hbm.at[brow + c], ssems[b])

    def compute_chunk(c, pbuf, obuf):
        @plsc.parallel_loop(0, CHUNK, unroll=UNROLL)
        def row_body(i):
            off = hoff_v[c, pl.ds(i, NLANE)][0]
            poff = i * D
            e = [pbuf[i, pl.ds(off + NLANE * k, NLANE)] +
                 pos_v[pl.ds(poff + NLANE * k, NLANE)]
                 for k in range(NVR)]
            s = (e[0] + e[1]) + (e[2] + e[3])
            q = (e[0] * e[0] + e[1] * e[1]) + (e[2] * e[2] + e[3] * e[3])
            mean = jnp.sum(s) * inv_d
            var = jnp.sum(q) * inv_d - mean * mean + jnp.float32(EPS)
            a = _srsqrt(var)
            nb = -mean * a
            av = jnp.full((NLANE,), a, dtype=jnp.float32)
            bv = jnp.full((NLANE,), nb, dtype=jnp.float32)
            for k in range(NVR):
                obuf[i, pl.ds(NLANE * k, NLANE)] = (e[k] * av + bv) * g[k] + bt[k]

    for d in gather(0, 0):
        d.start()

    def outer(gi, carry):
        for b in range(2):
            c = gi * 2 + b

            @pl.when(c <= NCHUNK - 2)
            def _():
                for d in gather(c + 1, 1 - b):
                    d.start()

            for d in gather(c, b):
                d.wait()

            @pl.when(c >= 2)
            def _():
                store(c - 2, b).wait()

            compute_chunk(c, pbufs[b], obufs[b])
            store(c, b).start()
        return carry

    lax.fori_loop(0, NCHUNK // 2, outer, 0)
    store(NCHUNK - 2, 0).wait()
    store(NCHUNK - 1, 1).wait()


@jax.jit
def _sc_embed(idxp, hoff, table2, pos_flat, gamma, beta):
    mesh = plsc.VectorSubcoreMesh(core_axis_name="c", subcore_axis_name="s")
    fn = pl.kernel(
        _body,
        out_type=jax.ShapeDtypeStruct((B, S, D), jnp.float32),
        mesh=mesh,
        compiler_params=pltpu.CompilerParams(
            needs_layout_passes=False, use_tc_tiling_on_sc=False
        ),
        scratch_types=[
            pltpu.VMEM((NCHUNK, CHUNK), jnp.int32),
            pltpu.VMEM((NCHUNK + 1, CHUNK), jnp.int32),
            pltpu.VMEM((S * D,), jnp.float32),
            [pltpu.VMEM((CHUNK, 2 * D), jnp.float32) for _ in range(2)],
            [pltpu.VMEM((CHUNK, D), jnp.float32) for _ in range(2)],
            pltpu.VMEM((D,), jnp.float32),
            pltpu.VMEM((D,), jnp.float32),
            [pltpu.SemaphoreType.DMA for _ in range(2)],
            [pltpu.SemaphoreType.DMA for _ in range(2)],
        ],
    )
    return fn(idxp, hoff, table2, pos_flat, gamma, beta)


def kernel(x, W_word, W_pos, gamma, beta):
    xi = x.astype(jnp.int32)
    table2 = _tc_repack(W_word.T)
    sb = xi >> 12
    r = xi & (2 * BKV - 1)
    idxp = (sb << 11) + (r & (BKV - 1))
    hoff = ((r >> 11) & 1) << 6
    return _sc_embed(idxp, hoff, table2, W_pos.reshape(S * D),
                     gamma.astype(jnp.float32), beta.astype(jnp.float32))


# speedup vs baseline: 1.3755x; 1.3755x over previous
import jax
import jax.numpy as jnp
from jax import lax
from jax.experimental import pallas as pl
from jax.experimental.pallas import tpu as pltpu
from jax.experimental.pallas import tpu_sc as plsc

B, S, D = 1024, 200, 64
VOCAB = 1000000
N = B * S
NC, NS = 2, 16
NW = NC * NS
NCHUNK = B // NW
CHUNK = S
G1 = 128
G2 = CHUNK - G1
EPS = 1e-12
NLANE = 16
NVR = D // NLANE
UNROLL = 8
BKV = 2048


def _srsqrt(v):
    i = lax.bitcast_convert_type(v, jnp.int32)
    y = lax.bitcast_convert_type(jnp.int32(0x5F3759DF) - (i >> 1), jnp.float32)
    hv = jnp.float32(0.5) * v
    for _ in range(3):
        y = y * (jnp.float32(1.5) - hv * y * y)
    return y


NSB = (VOCAB + 2 * BKV - 1) // (2 * BKV)
VROWS = NSB * BKV


def _repack_body(wt1_ref, wt2_ref, out_ref):
    out_ref[...] = jnp.concatenate([wt1_ref[...].T, wt2_ref[...].T], axis=1)


@jax.jit
def _tc_repack(Wt):
    return pl.pallas_call(
        _repack_body,
        grid=(NSB,),
        in_specs=[pl.BlockSpec((D, BKV), lambda i: (0, 2 * i)),
                  pl.BlockSpec(
                      (D, BKV),
                      lambda i: (0, jnp.minimum(2 * i + 1,
                                                (VOCAB - 1) // BKV)))],
        out_specs=pl.BlockSpec((BKV, 2 * D), lambda i: (i, 0)),
        out_shape=jax.ShapeDtypeStruct((VROWS, 2 * D), jnp.float32),
    )(Wt, Wt)


def _body(idxp_hbm, hoff_hbm, table2_hbm, pos_hbm, gamma_hbm, beta_hbm,
          out3_hbm, idxp_v, hoff_v, pos_v, pbufs, obufs, gam_v, bet_v,
          gsems, ssems):
    wid = lax.axis_index("s") * NC + lax.axis_index("c")
    brow = wid * NCHUNK
    pltpu.sync_copy(idxp_hbm.at[pl.ds(brow, NCHUNK)], idxp_v)
    pltpu.sync_copy(hoff_hbm.at[pl.ds(brow, NCHUNK)],
                    hoff_v.at[pl.ds(0, NCHUNK)])
    pltpu.sync_copy(pos_hbm, pos_v)
    pltpu.sync_copy(gamma_hbm, gam_v)
    pltpu.sync_copy(beta_hbm, bet_v)
    g = [gam_v[pl.ds(NLANE * k, NLANE)] for k in range(NVR)]
    bt = [bet_v[pl.ds(NLANE * k, NLANE)] for k in range(NVR)]
    inv_d = jnp.float32(1.0 / D)

    def gather(c, b):
        return (
            pltpu.make_async_copy(
                table2_hbm.at[idxp_v.at[c, pl.ds(0, G1)]],
                pbufs[b].at[pl.ds(0, G1)], gsems[b]),
            pltpu.make_async_copy(
                table2_hbm.at[idxp_v.at[c, pl.ds(G1, G2)]],
                pbufs[b].at[pl.ds(G1, G2)], gsems[b]),
        )

    def store(c, b):
        return pltpu.make_async_copy(obufs[b], out3_hbm.at[brow + c], ssems[b])

    def compute_chunk(c, pbuf, obuf):
        @plsc.parallel_loop(0, CHUNK, unroll=UNROLL)
        def row_body(i):
            off = hoff_v[c, pl.ds(i, NLANE)][0]
            poff = i * D
            e = [pbuf[i, pl.ds(off + NLANE * k, NLANE)] +
                 pos_v[pl.ds(poff + NLANE * k, NLANE)]
                 for k in range(NVR)]
            s = (e[0] + e[1]) + (e[2] + e[3])
            q = (e[0] * e[0] + e[1] * e[1]) + (e[2] * e[2] + e[3] * e[3])
            mean = jnp.sum(s) * inv_d
            var = jnp.sum(q) * inv_d - mean * mean + jnp.float32(EPS)
            a = _srsqrt(var)
            nb = -mean * a
            av = jnp.full((NLANE,), a, dtype=jnp.float32)
            bv = jnp.full((NLANE,), nb, dtype=jnp.float32)
            for k in range(NVR):
                obuf[i, pl.ds(NLANE * k, NLANE)] = (e[k] * av + bv) * g[k] + bt[k]

    for d in gather(0, 0):
        d.start()

    def outer(gi, carry):
        for b in range(2):
            c = gi * 2 + b

            @pl.when(c <= NCHUNK - 2)
            def _():
                for d in gather(c + 1, 1 - b):
                    d.start()

            for d in gather(c, b):
                d.wait()

            @pl.when(c >= 2)
            def _():
                store(c - 2, b).wait()

            compute_chunk(c, pbufs[b], obufs[b])
            store(c, b).start()
        return carry

    lax.fori_loop(0, NCHUNK // 2, outer, 0)
    store(NCHUNK - 2, 0).wait()
    store(NCHUNK - 1, 1).wait()


@jax.jit
def _sc_embed(idxp, hoff, table2, pos_flat, gamma, beta):
    mesh = plsc.VectorSubcoreMesh(core_axis_name="c", subcore_axis_name="s")
    fn = pl.kernel(
        _body,
        out_type=jax.ShapeDtypeStruct((B, S, D), jnp.float32),
        mesh=mesh,
        compiler_params=pltpu.CompilerParams(
            needs_layout_passes=False, use_tc_tiling_on_sc=False
        ),
        scratch_types=[
            pltpu.VMEM((NCHUNK, CHUNK), jnp.int32),
            pltpu.VMEM((NCHUNK + 1, CHUNK), jnp.int32),
            pltpu.VMEM((S * D,), jnp.float32),
            [pltpu.VMEM((CHUNK, 2 * D), jnp.float32) for _ in range(2)],
            [pltpu.VMEM((CHUNK, D), jnp.float32) for _ in range(2)],
            pltpu.VMEM((D,), jnp.float32),
            pltpu.VMEM((D,), jnp.float32),
            [pltpu.SemaphoreType.DMA for _ in range(2)],
            [pltpu.SemaphoreType.DMA for _ in range(2)],
        ],
    )
    return fn(idxp, hoff, table2, pos_flat, gamma, beta)


def kernel(x, W_word, W_pos, gamma, beta):
    xi = x.astype(jnp.int32)
    table2 = _tc_repack(W_word.T)
    sb = xi >> 12
    r = xi & (2 * BKV - 1)
    idxp = (sb << 11) + (r & (BKV - 1))
    hoff = ((r >> 11) & 1) << 6
    return _sc_embed(idxp, hoff, table2, W_pos.reshape(S * D),
                     gamma.astype(jnp.float32), beta.astype(jnp.float32))


# speedup vs baseline: 1.5683x; 1.1402x over previous
import jax
import jax.numpy as jnp
from jax import lax
from jax.experimental import pallas as pl
from jax.experimental.pallas import tpu as pltpu
from jax.experimental.pallas import tpu_sc as plsc

B, S, D = 1024, 200, 64
VOCAB = 1000000
N = B * S
NC, NS = 2, 16
NW = NC * NS
NCHUNK = B // NW
CHUNK = S
G1 = 128
G2 = CHUNK - G1
EPS = 1e-12
NLANE = 16
NVR = D // NLANE
UNROLL = 8
BKV = 4096
BKV_SH = 12


def _srsqrt(v):
    i = lax.bitcast_convert_type(v, jnp.int32)
    y = lax.bitcast_convert_type(jnp.int32(0x5F3759DF) - (i >> 1), jnp.float32)
    hv = jnp.float32(0.5) * v
    for _ in range(3):
        y = y * (jnp.float32(1.5) - hv * y * y)
    return y


NSB = (VOCAB + 2 * BKV - 1) // (2 * BKV)
VROWS = NSB * BKV


def _repack_body(wt1_ref, wt2_ref, out_ref):
    out_ref[...] = jnp.concatenate([wt1_ref[...].T, wt2_ref[...].T], axis=1)


@jax.jit
def _tc_repack(Wt):
    return pl.pallas_call(
        _repack_body,
        grid=(NSB,),
        in_specs=[pl.BlockSpec((D, BKV), lambda i: (0, 2 * i)),
                  pl.BlockSpec(
                      (D, BKV),
                      lambda i: (0, jnp.minimum(2 * i + 1,
                                                (VOCAB - 1) // BKV)))],
        out_specs=pl.BlockSpec((BKV, 2 * D), lambda i: (i, 0)),
        out_shape=jax.ShapeDtypeStruct((VROWS, 2 * D), jnp.float32),
    )(Wt, Wt)


def _body(idxp_hbm, hoff_hbm, table2_hbm, pos_hbm, gamma_hbm, beta_hbm,
          out3_hbm, idxp_v, hoff_v, pos_v, pbufs, obufs, gam_v, bet_v,
          gsems, ssems):
    wid = lax.axis_index("s") * NC + lax.axis_index("c")
    brow = wid * NCHUNK
    pltpu.sync_copy(idxp_hbm.at[pl.ds(brow, NCHUNK)], idxp_v)
    pltpu.sync_copy(hoff_hbm.at[pl.ds(brow, NCHUNK)],
                    hoff_v.at[pl.ds(0, NCHUNK)])
    pltpu.sync_copy(pos_hbm, pos_v)
    pltpu.sync_copy(gamma_hbm, gam_v)
    pltpu.sync_copy(beta_hbm, bet_v)
    g = [gam_v[pl.ds(NLANE * k, NLANE)] for k in range(NVR)]
    bt = [bet_v[pl.ds(NLANE * k, NLANE)] for k in range(NVR)]
    inv_d = jnp.float32(1.0 / D)

    def gather(c, b):
        return (
            pltpu.make_async_copy(
                table2_hbm.at[idxp_v.at[c, pl.ds(0, G1)]],
                pbufs[b].at[pl.ds(0, G1)], gsems[b]),
            pltpu.make_async_copy(
                table2_hbm.at[idxp_v.at[c, pl.ds(G1, G2)]],
                pbufs[b].at[pl.ds(G1, G2)], gsems[b]),
        )

    def store(c, b):
        return pltpu.make_async_copy(obufs[b], out3_hbm.at[brow + c], ssems[b])

    def compute_chunk(c, pbuf, obuf):
        @plsc.parallel_loop(0, CHUNK, unroll=UNROLL)
        def row_body(i):
            off = hoff_v[c, pl.ds(i, NLANE)][0]
            poff = i * D
            e = [pbuf[i, pl.ds(off + NLANE * k, NLANE)] +
                 pos_v[pl.ds(poff + NLANE * k, NLANE)]
                 for k in range(NVR)]
            s = (e[0] + e[1]) + (e[2] + e[3])
            q = (e[0] * e[0] + e[1] * e[1]) + (e[2] * e[2] + e[3] * e[3])
            mean = jnp.sum(s) * inv_d
            var = jnp.sum(q) * inv_d - mean * mean + jnp.float32(EPS)
            a = _srsqrt(var)
            nb = -mean * a
            av = jnp.full((NLANE,), a, dtype=jnp.float32)
            bv = jnp.full((NLANE,), nb, dtype=jnp.float32)
            for k in range(NVR):
                obuf[i, pl.ds(NLANE * k, NLANE)] = (e[k] * av + bv) * g[k] + bt[k]

    for d in gather(0, 0):
        d.start()

    def outer(gi, carry):
        for b in range(2):
            c = gi * 2 + b

            @pl.when(c <= NCHUNK - 2)
            def _():
                for d in gather(c + 1, 1 - b):
                    d.start()

            for d in gather(c, b):
                d.wait()

            @pl.when(c >= 2)
            def _():
                store(c - 2, b).wait()

            compute_chunk(c, pbufs[b], obufs[b])
            store(c, b).start()
        return carry

    lax.fori_loop(0, NCHUNK // 2, outer, 0)
    store(NCHUNK - 2, 0).wait()
    store(NCHUNK - 1, 1).wait()


@jax.jit
def _sc_embed(idxp, hoff, table2, pos_flat, gamma, beta):
    mesh = plsc.VectorSubcoreMesh(core_axis_name="c", subcore_axis_name="s")
    fn = pl.kernel(
        _body,
        out_type=jax.ShapeDtypeStruct((B, S, D), jnp.float32),
        mesh=mesh,
        compiler_params=pltpu.CompilerParams(
            needs_layout_passes=False, use_tc_tiling_on_sc=False
        ),
        scratch_types=[
            pltpu.VMEM((NCHUNK, CHUNK), jnp.int32),
            pltpu.VMEM((NCHUNK + 1, CHUNK), jnp.int32),
            pltpu.VMEM((S * D,), jnp.float32),
            [pltpu.VMEM((CHUNK, 2 * D), jnp.float32) for _ in range(2)],
            [pltpu.VMEM((CHUNK, D), jnp.float32) for _ in range(2)],
            pltpu.VMEM((D,), jnp.float32),
            pltpu.VMEM((D,), jnp.float32),
            [pltpu.SemaphoreType.DMA for _ in range(2)],
            [pltpu.SemaphoreType.DMA for _ in range(2)],
        ],
    )
    return fn(idxp, hoff, table2, pos_flat, gamma, beta)


def kernel(x, W_word, W_pos, gamma, beta):
    xi = x.astype(jnp.int32)
    table2 = _tc_repack(W_word.T)
    sb = xi >> (BKV_SH + 1)
    r = xi & (2 * BKV - 1)
    idxp = (sb << BKV_SH) + (r & (BKV - 1))
    hoff = ((r >> BKV_SH) & 1) << 6
    return _sc_embed(idxp, hoff, table2, W_pos.reshape(S * D),
                     gamma.astype(jnp.float32), beta.astype(jnp.float32))


# speedup vs baseline: 1.6808x; 1.0717x over previous
import jax
import jax.numpy as jnp
from jax import lax
from jax.experimental import pallas as pl
from jax.experimental.pallas import tpu as pltpu
from jax.experimental.pallas import tpu_sc as plsc

B, S, D = 1024, 200, 64
VOCAB = 1000000
N = B * S
NC, NS = 2, 16
NW = NC * NS
NCHUNK = B // NW
CHUNK = S
G1 = 128
G2 = CHUNK - G1
EPS = 1e-12
NLANE = 16
NVR = D // NLANE
UNROLL = 8
BKV = 8192
BKV_SH = 13


def _srsqrt(v):
    i = lax.bitcast_convert_type(v, jnp.int32)
    y = lax.bitcast_convert_type(jnp.int32(0x5F3759DF) - (i >> 1), jnp.float32)
    hv = jnp.float32(0.5) * v
    for _ in range(3):
        y = y * (jnp.float32(1.5) - hv * y * y)
    return y


NSB = (VOCAB + 2 * BKV - 1) // (2 * BKV)
VROWS = NSB * BKV


def _repack_body(wt1_ref, wt2_ref, out_ref):
    out_ref[...] = jnp.concatenate([wt1_ref[...].T, wt2_ref[...].T], axis=1)


@jax.jit
def _tc_repack(Wt):
    return pl.pallas_call(
        _repack_body,
        grid=(NSB,),
        in_specs=[pl.BlockSpec((D, BKV), lambda i: (0, 2 * i)),
                  pl.BlockSpec(
                      (D, BKV),
                      lambda i: (0, jnp.minimum(2 * i + 1,
                                                (VOCAB - 1) // BKV)))],
        out_specs=pl.BlockSpec((BKV, 2 * D), lambda i: (i, 0)),
        out_shape=jax.ShapeDtypeStruct((VROWS, 2 * D), jnp.float32),
    )(Wt, Wt)


def _body(idxp_hbm, hoff_hbm, table2_hbm, pos_hbm, gamma_hbm, beta_hbm,
          out3_hbm, idxp_v, hoff_v, pos_v, pbufs, obufs, gam_v, bet_v,
          gsems, ssems):
    wid = lax.axis_index("s") * NC + lax.axis_index("c")
    brow = wid * NCHUNK
    pltpu.sync_copy(idxp_hbm.at[pl.ds(brow, NCHUNK)], idxp_v)
    pltpu.sync_copy(hoff_hbm.at[pl.ds(brow, NCHUNK)],
                    hoff_v.at[pl.ds(0, NCHUNK)])
    pltpu.sync_copy(pos_hbm, pos_v)
    pltpu.sync_copy(gamma_hbm, gam_v)
    pltpu.sync_copy(beta_hbm, bet_v)
    g = [gam_v[pl.ds(NLANE * k, NLANE)] for k in range(NVR)]
    bt = [bet_v[pl.ds(NLANE * k, NLANE)] for k in range(NVR)]
    inv_d = jnp.float32(1.0 / D)

    def gather(c, b):
        return (
            pltpu.make_async_copy(
                table2_hbm.at[idxp_v.at[c, pl.ds(0, G1)]],
                pbufs[b].at[pl.ds(0, G1)], gsems[b]),
            pltpu.make_async_copy(
                table2_hbm.at[idxp_v.at[c, pl.ds(G1, G2)]],
                pbufs[b].at[pl.ds(G1, G2)], gsems[b]),
        )

    def store(c, b):
        return pltpu.make_async_copy(obufs[b], out3_hbm.at[brow + c], ssems[b])

    def compute_chunk(c, pbuf, obuf):
        @plsc.parallel_loop(0, CHUNK, unroll=UNROLL)
        def row_body(i):
            off = hoff_v[c, pl.ds(i, NLANE)][0]
            poff = i * D
            e = [pbuf[i, pl.ds(off + NLANE * k, NLANE)] +
                 pos_v[pl.ds(poff + NLANE * k, NLANE)]
                 for k in range(NVR)]
            s = (e[0] + e[1]) + (e[2] + e[3])
            q = (e[0] * e[0] + e[1] * e[1]) + (e[2] * e[2] + e[3] * e[3])
            mean = jnp.sum(s) * inv_d
            var = jnp.sum(q) * inv_d - mean * mean + jnp.float32(EPS)
            a = _srsqrt(var)
            nb = -mean * a
            av = jnp.full((NLANE,), a, dtype=jnp.float32)
            bv = jnp.full((NLANE,), nb, dtype=jnp.float32)
            for k in range(NVR):
                obuf[i, pl.ds(NLANE * k, NLANE)] = (e[k] * av + bv) * g[k] + bt[k]

    for d in gather(0, 0):
        d.start()

    def outer(gi, carry):
        for b in range(2):
            c = gi * 2 + b

            @pl.when(c <= NCHUNK - 2)
            def _():
                for d in gather(c + 1, 1 - b):
                    d.start()

            for d in gather(c, b):
                d.wait()

            @pl.when(c >= 2)
            def _():
                store(c - 2, b).wait()

            compute_chunk(c, pbufs[b], obufs[b])
            store(c, b).start()
        return carry

    lax.fori_loop(0, NCHUNK // 2, outer, 0)
    store(NCHUNK - 2, 0).wait()
    store(NCHUNK - 1, 1).wait()


@jax.jit
def _sc_embed(idxp, hoff, table2, pos_flat, gamma, beta):
    mesh = plsc.VectorSubcoreMesh(core_axis_name="c", subcore_axis_name="s")
    fn = pl.kernel(
        _body,
        out_type=jax.ShapeDtypeStruct((B, S, D), jnp.float32),
        mesh=mesh,
        compiler_params=pltpu.CompilerParams(
            needs_layout_passes=False, use_tc_tiling_on_sc=False
        ),
        scratch_types=[
            pltpu.VMEM((NCHUNK, CHUNK), jnp.int32),
            pltpu.VMEM((NCHUNK + 1, CHUNK), jnp.int32),
            pltpu.VMEM((S * D,), jnp.float32),
            [pltpu.VMEM((CHUNK, 2 * D), jnp.float32) for _ in range(2)],
            [pltpu.VMEM((CHUNK, D), jnp.float32) for _ in range(2)],
            pltpu.VMEM((D,), jnp.float32),
            pltpu.VMEM((D,), jnp.float32),
            [pltpu.SemaphoreType.DMA for _ in range(2)],
            [pltpu.SemaphoreType.DMA for _ in range(2)],
        ],
    )
    return fn(idxp, hoff, table2, pos_flat, gamma, beta)


def kernel(x, W_word, W_pos, gamma, beta):
    xi = x.astype(jnp.int32)
    table2 = _tc_repack(W_word.T)
    sb = xi >> (BKV_SH + 1)
    r = xi & (2 * BKV - 1)
    idxp = (sb << BKV_SH) + (r & (BKV - 1))
    hoff = ((r >> BKV_SH) & 1) << 6
    return _sc_embed(idxp, hoff, table2, W_pos.reshape(S * D),
                     gamma.astype(jnp.float32), beta.astype(jnp.float32))
